# baseline (speedup 1.0000x reference)
import functools

import jax
import jax.numpy as jnp
from jax import lax
from jax.experimental import pallas as pl
from jax.experimental.pallas import tpu as pltpu
from jax.experimental.pallas import tpu_sc as plsc

NUM_CORES = 2
NUM_SUBCORES = 16
NUM_WORKERS = NUM_CORES * NUM_SUBCORES
CHUNK = 128
NBUF = 5
GATHER_ONLY = True


@functools.partial(jax.jit, static_argnums=(2, 3))
def _gather_flat(idx3d, table, b_per_w, n_chunks):
    D = table.shape[1]
    B = NUM_WORKERS * b_per_w
    n_outer = n_chunks // NBUF
    mesh = plsc.VectorSubcoreMesh(core_axis_name="c", subcore_axis_name="s")

    @functools.partial(
        pl.kernel,
        out_type=jax.ShapeDtypeStruct((B, D), jnp.float32),
        mesh=mesh,
        scratch_types=[
            pltpu.VMEM((n_chunks, CHUNK), jnp.int32),
            pltpu.VMEM((NBUF, CHUNK, D), jnp.float32),
            pltpu.SemaphoreType.DMA((NBUF,)),
            pltpu.SemaphoreType.DMA((NBUF,)),
        ],
    )
    def k(idx_hbm, table_hbm, out_hbm, idx_v, bufs, gsem, osem):
        wid = lax.axis_index("s") * NUM_CORES + lax.axis_index("c")
        base = wid * b_per_w
        pltpu.sync_copy(idx_hbm.at[wid], idx_v)

        def outer(tt, carry):
            for b in range(NBUF):
                row = tt * NBUF + b

                if not GATHER_ONLY:
                    @pl.when(tt > 0)
                    def _wait_out(b=b):
                        pltpu.make_async_copy(
                            bufs.at[b], out_hbm.at[pl.ds(base, CHUNK)], osem.at[b]
                        ).wait()

                pltpu.async_copy(
                    table_hbm.at[idx_v.at[row]], bufs.at[b], gsem.at[b]
                )
            for b in range(NBUF):
                row = tt * NBUF + b
                pltpu.make_async_copy(
                    table_hbm.at[idx_v.at[row]], bufs.at[b], gsem.at[b]
                ).wait()
                if not GATHER_ONLY:
                    pltpu.async_copy(
                        bufs.at[b],
                        out_hbm.at[pl.ds(base + row * CHUNK, CHUNK)],
                        osem.at[b],
                    )
            return carry

        lax.fori_loop(0, n_outer, outer, 0)
        if not GATHER_ONLY:
            for b in range(NBUF):
                pltpu.make_async_copy(
                    bufs.at[b], out_hbm.at[pl.ds(base, CHUNK)], osem.at[b]
                ).wait()
        else:
            pltpu.sync_copy(bufs.at[0], out_hbm.at[pl.ds(base, CHUNK)])

    return k(idx3d, table)


def kernel(inputs, table):
    B0, H = inputs.shape
    D = table.shape[1]
    B = B0 * H
    assert B % (NUM_WORKERS * CHUNK) == 0
    b_per_w = B // NUM_WORKERS
    n_chunks = b_per_w // CHUNK
    idx3d = jnp.zeros_like(inputs, jnp.int32).reshape(NUM_WORKERS, n_chunks, CHUNK)
    out = _gather_flat(idx3d, table, b_per_w, n_chunks)
    return out.reshape(B0, H, D)


# speedup vs baseline: 27.0085x; 27.0085x over previous
import functools

import jax
import jax.numpy as jnp
from jax import lax
from jax.experimental import pallas as pl
from jax.experimental.pallas import tpu as pltpu
from jax.experimental.pallas import tpu_sc as plsc

NUM_CORES = 2
NUM_SUBCORES = 16
NUM_WORKERS = NUM_CORES * NUM_SUBCORES
CHUNK = 128
NBUF = 5


@functools.partial(jax.jit, static_argnums=(2, 3))
def _gather_flat(idx3d, table, b_per_w, n_chunks):
    D = table.shape[1]
    B = NUM_WORKERS * b_per_w
    n_outer = n_chunks // NBUF
    mesh = plsc.VectorSubcoreMesh(core_axis_name="c", subcore_axis_name="s")

    @functools.partial(
        pl.kernel,
        out_type=jax.ShapeDtypeStruct((B, D), jnp.float32),
        mesh=mesh,
        scratch_types=[
            pltpu.VMEM((n_chunks, CHUNK), jnp.int32),
            pltpu.VMEM((NBUF, CHUNK, D), jnp.float32),
            pltpu.SemaphoreType.DMA((NBUF,)),
            pltpu.SemaphoreType.DMA((NBUF,)),
        ],
    )
    def k(idx_hbm, table_hbm, out_hbm, idx_v, bufs, gsem, osem):
        wid = lax.axis_index("s") * NUM_CORES + lax.axis_index("c")
        base = wid * b_per_w
        pltpu.sync_copy(idx_hbm.at[wid], idx_v)

        def outer(tt, carry):
            for b in range(NBUF):
                row = tt * NBUF + b

                @pl.when(tt > 0)
                def _wait_out(b=b):
                    pltpu.make_async_copy(
                        bufs.at[b], out_hbm.at[pl.ds(base, CHUNK)], osem.at[b]
                    ).wait()

                pltpu.async_copy(
                    table_hbm.at[idx_v.at[row]], bufs.at[b], gsem.at[b]
                )
            for b in range(NBUF):
                row = tt * NBUF + b
                pltpu.make_async_copy(
                    table_hbm.at[idx_v.at[row]], bufs.at[b], gsem.at[b]
                ).wait()
                pltpu.async_copy(
                    bufs.at[b],
                    out_hbm.at[pl.ds(base + row * CHUNK, CHUNK)],
                    osem.at[b],
                )
            return carry

        lax.fori_loop(0, n_outer, outer, 0)
        for b in range(NBUF):
            pltpu.make_async_copy(
                bufs.at[b], out_hbm.at[pl.ds(base, CHUNK)], osem.at[b]
            ).wait()

    return k(idx3d, table)


def kernel(inputs, table):
    B0, H = inputs.shape
    D = table.shape[1]
    B = B0 * H
    assert B % (NUM_WORKERS * CHUNK) == 0
    b_per_w = B // NUM_WORKERS
    n_chunks = b_per_w // CHUNK
    idx3d = inputs.astype(jnp.int32).reshape(NUM_WORKERS, n_chunks, CHUNK)
    out = _gather_flat(idx3d, table, b_per_w, n_chunks)
    return out.reshape(B0, H, D)


# speedup vs baseline: 48.2755x; 1.7874x over previous
import functools

import jax
import jax.numpy as jnp
from jax import lax
from jax.experimental import pallas as pl
from jax.experimental.pallas import tpu as pltpu
from jax.experimental.pallas import tpu_sc as plsc

NUM_CORES = 2
NUM_SUBCORES = 16
NUM_WORKERS = NUM_CORES * NUM_SUBCORES
NBUF = 8


@jax.jit
def _embed_lookup(idx, table):
    B0, H = idx.shape
    D = table.shape[1]
    s_per_w = B0 // NUM_WORKERS
    n_outer = s_per_w // NBUF
    mesh = plsc.VectorSubcoreMesh(core_axis_name="c", subcore_axis_name="s")

    @functools.partial(
        pl.kernel,
        out_type=jax.ShapeDtypeStruct((B0, H, D), jnp.float32),
        mesh=mesh,
        scratch_types=[
            pltpu.VMEM((s_per_w, H), jnp.int32),
            pltpu.VMEM((NBUF, H, D), jnp.float32),
            pltpu.SemaphoreType.DMA((NBUF,)),
            pltpu.SemaphoreType.DMA((NBUF,)),
        ],
    )
    def k(idx_hbm, table_hbm, out_hbm, idx_v, bufs, gsem, osem):
        wid = lax.axis_index("s") * NUM_CORES + lax.axis_index("c")
        base = wid * s_per_w
        pltpu.sync_copy(idx_hbm.at[pl.ds(base, s_per_w)], idx_v)

        def outer(tt, carry):
            for b in range(NBUF):
                sl = tt * NBUF + b

                @pl.when(tt > 0)
                def _wait_out(b=b):
                    pltpu.make_async_copy(
                        bufs.at[b], out_hbm.at[base], osem.at[b]
                    ).wait()

                pltpu.async_copy(
                    table_hbm.at[idx_v.at[sl]], bufs.at[b], gsem.at[b]
                )
            for b in range(NBUF):
                sl = tt * NBUF + b
                pltpu.make_async_copy(
                    table_hbm.at[idx_v.at[sl]], bufs.at[b], gsem.at[b]
                ).wait()
                pltpu.async_copy(bufs.at[b], out_hbm.at[base + sl], osem.at[b])
            return carry

        lax.fori_loop(0, n_outer, outer, 0)
        for b in range(NBUF):
            pltpu.make_async_copy(
                bufs.at[b], out_hbm.at[base], osem.at[b]
            ).wait()

    return k(idx, table)


def kernel(inputs, table):
    return _embed_lookup(inputs.astype(jnp.int32), table)


# speedup vs baseline: 84.9417x; 1.7595x over previous
import functools

import jax
import jax.numpy as jnp
from jax import lax
from jax.experimental import pallas as pl
from jax.experimental.pallas import tpu as pltpu
from jax.experimental.pallas import tpu_sc as plsc

NUM_CORES = 2
NUM_SUBCORES = 16
NUM_WORKERS = NUM_CORES * NUM_SUBCORES
NBUF = 5


@jax.jit
def _embed_lookup(idx_t, table):
    H, B0 = idx_t.shape
    D = table.shape[1]
    s_per_w = B0 // NUM_WORKERS
    n_outer = H // NBUF
    mesh = plsc.VectorSubcoreMesh(core_axis_name="c", subcore_axis_name="s")

    @functools.partial(
        pl.kernel,
        out_type=jax.ShapeDtypeStruct((H, B0, D), jnp.float32),
        mesh=mesh,
        scratch_types=[
            pltpu.VMEM((H, s_per_w), jnp.int32),
            pltpu.VMEM((NBUF, s_per_w, D), jnp.float32),
            pltpu.SemaphoreType.DMA((NBUF,)),
            pltpu.SemaphoreType.DMA((NBUF,)),
        ],
    )
    def k(idx_hbm, table_hbm, out_hbm, idx_v, bufs, gsem, osem):
        wid = lax.axis_index("s") * NUM_CORES + lax.axis_index("c")
        base = wid * s_per_w
        pltpu.sync_copy(idx_hbm.at[:, pl.ds(base, s_per_w)], idx_v)

        def outer(tt, carry):
            for b in range(NBUF):
                h = tt * NBUF + b

                @pl.when(tt > 0)
                def _wait_out(b=b):
                    pltpu.make_async_copy(
                        bufs.at[b],
                        out_hbm.at[0, pl.ds(base, s_per_w)],
                        osem.at[b],
                    ).wait()

                pltpu.async_copy(
                    table_hbm.at[idx_v.at[h]], bufs.at[b], gsem.at[b]
                )
            for b in range(NBUF):
                h = tt * NBUF + b
                pltpu.make_async_copy(
                    table_hbm.at[idx_v.at[h]], bufs.at[b], gsem.at[b]
                ).wait()
                pltpu.async_copy(
                    bufs.at[b],
                    out_hbm.at[h, pl.ds(base, s_per_w)],
                    osem.at[b],
                )
            return carry

        lax.fori_loop(0, n_outer, outer, 0)
        for b in range(NBUF):
            pltpu.make_async_copy(
                bufs.at[b], out_hbm.at[0, pl.ds(base, s_per_w)], osem.at[b]
            ).wait()

    return k(idx_t, table)


def kernel(inputs, table):
    out_t = _embed_lookup(inputs.astype(jnp.int32).T, table)
    return out_t.transpose(1, 0, 2)


# speedup vs baseline: 87.4736x; 1.0298x over previous
import functools

import jax
import jax.numpy as jnp
from jax import lax
from jax.experimental import pallas as pl
from jax.experimental.pallas import tpu as pltpu
from jax.experimental.pallas import tpu_sc as plsc

NUM_CORES = 2
NUM_SUBCORES = 16
NUM_WORKERS = NUM_CORES * NUM_SUBCORES
NBUF = 10
SPLIT = 2


@jax.jit
def _embed_lookup(idx_t, table):
    H, B0 = idx_t.shape
    D = table.shape[1]
    s_per_w = B0 // NUM_WORKERS
    n_chunks = H * SPLIT
    n_outer = n_chunks // NBUF
    c_sz = s_per_w // SPLIT
    mesh = plsc.VectorSubcoreMesh(core_axis_name="c", subcore_axis_name="s")

    @functools.partial(
        pl.kernel,
        out_type=jax.ShapeDtypeStruct((H, B0, D), jnp.float32),
        mesh=mesh,
        scratch_types=[
            pltpu.VMEM((H, s_per_w), jnp.int32),
            pltpu.VMEM((NBUF, c_sz, D), jnp.float32),
            pltpu.SemaphoreType.DMA((NBUF,)),
            pltpu.SemaphoreType.DMA((NBUF,)),
        ],
    )
    def k(idx_hbm, table_hbm, out_hbm, idx_v, bufs, gsem, osem):
        wid = lax.axis_index("s") * NUM_CORES + lax.axis_index("c")
        base = wid * s_per_w
        pltpu.sync_copy(idx_hbm.at[:, pl.ds(base, s_per_w)], idx_v)

        def outer(tt, carry):
            for b in range(NBUF):
                h = tt * (NBUF // SPLIT) + b // SPLIT
                off = (b % SPLIT) * c_sz

                @pl.when(tt > 0)
                def _wait_out(b=b):
                    pltpu.make_async_copy(
                        bufs.at[b],
                        out_hbm.at[0, pl.ds(base, c_sz)],
                        osem.at[b],
                    ).wait()

                pltpu.async_copy(
                    table_hbm.at[idx_v.at[h, pl.ds(off, c_sz)]],
                    bufs.at[b],
                    gsem.at[b],
                )
            for b in range(NBUF):
                h = tt * (NBUF // SPLIT) + b // SPLIT
                off = (b % SPLIT) * c_sz
                pltpu.make_async_copy(
                    table_hbm.at[idx_v.at[h, pl.ds(off, c_sz)]],
                    bufs.at[b],
                    gsem.at[b],
                ).wait()
                pltpu.async_copy(
                    bufs.at[b],
                    out_hbm.at[h, pl.ds(base + off, c_sz)],
                    osem.at[b],
                )
            return carry

        lax.fori_loop(0, n_outer, outer, 0)
        for b in range(NBUF):
            pltpu.make_async_copy(
                bufs.at[b], out_hbm.at[0, pl.ds(base, c_sz)], osem.at[b]
            ).wait()

    return k(idx_t, table)


def kernel(inputs, table):
    out_t = _embed_lookup(inputs.astype(jnp.int32).T, table)
    return out_t.transpose(1, 0, 2)
